# baseline (speedup 1.0000x reference)
import jax
import jax.numpy as jnp
from jax.experimental import pallas as pl
from jax.experimental.pallas import tpu as pltpu

_K_PAD = 512
_N_PAD = 128


def _fused_kernel(x_ref, w_ref, b_ref, o_ref):
    x = jnp.maximum(x_ref[...], jnp.bfloat16(0))
    acc = jnp.dot(x, w_ref[...], preferred_element_type=jnp.float32)
    o_ref[...] = acc + b_ref[...]


@jax.jit
def kernel(feat, w, b):
    B, D = feat.shape
    _, N = w.shape

    tb = min(4096, max(8, (B + 7) // 8 * 8))
    b_pad = (B + tb - 1) // tb * tb

    feat_p = jnp.pad(feat.astype(jnp.bfloat16),
                     ((0, b_pad - B), (0, _K_PAD - D)))
    w_p = jnp.pad(w, ((0, _K_PAD - D), (0, _N_PAD - N))).astype(jnp.bfloat16)
    b_p = jnp.pad(b.reshape(1, N), ((0, 0), (0, _N_PAD - N)))

    out = pl.pallas_call(
        _fused_kernel,
        out_shape=jax.ShapeDtypeStruct((b_pad, _N_PAD), jnp.float32),
        grid=(b_pad // tb,),
        in_specs=[
            pl.BlockSpec((tb, _K_PAD), lambda i: (i, 0)),
            pl.BlockSpec((_K_PAD, _N_PAD), lambda i: (0, 0)),
            pl.BlockSpec((1, _N_PAD), lambda i: (0, 0)),
        ],
        out_specs=pl.BlockSpec((tb, _N_PAD), lambda i: (i, 0)),
        compiler_params=pltpu.CompilerParams(
            dimension_semantics=("arbitrary",),
        ),
    )(feat_p, w_p, b_p)

    return out[:B, :N]


# speedup vs baseline: 1.2371x; 1.2371x over previous
import jax
import jax.numpy as jnp
from jax.experimental import pallas as pl
from jax.experimental.pallas import tpu as pltpu


def _fused_kernel(x_ref, w_ref, b_ref, o_ref):
    x = jnp.maximum(x_ref[...], 0.0)
    acc = jnp.dot(x, w_ref[...], preferred_element_type=jnp.float32)
    o_ref[...] = acc + b_ref[...]


@jax.jit
def kernel(feat, w, b):
    B, D = feat.shape
    _, N = w.shape

    tb = min(8192, max(8, (B + 7) // 8 * 8))
    b_pad = (B + tb - 1) // tb * tb
    feat_p = jnp.pad(feat, ((0, b_pad - B), (0, 0))) if b_pad != B else feat

    out = pl.pallas_call(
        _fused_kernel,
        out_shape=jax.ShapeDtypeStruct((b_pad, N), jnp.float32),
        grid=(b_pad // tb,),
        in_specs=[
            pl.BlockSpec((tb, D), lambda i: (i, 0)),
            pl.BlockSpec((D, N), lambda i: (0, 0)),
            pl.BlockSpec((1, N), lambda i: (0, 0)),
        ],
        out_specs=pl.BlockSpec((tb, N), lambda i: (i, 0)),
        compiler_params=pltpu.CompilerParams(
            dimension_semantics=("arbitrary",),
        ),
    )(feat_p, w, b.reshape(1, N))

    return out[:B]


# speedup vs baseline: 1.2542x; 1.0137x over previous
import jax
import jax.numpy as jnp
from jax.experimental import pallas as pl
from jax.experimental.pallas import tpu as pltpu


def _fused_kernel(x_ref, w_ref, b_ref, o_ref):
    x = jnp.maximum(x_ref[...], 0.0)
    acc = jnp.dot(x, w_ref[...], preferred_element_type=jnp.float32)
    o_ref[...] = acc + b_ref[...]


@jax.jit
def kernel(feat, w, b):
    B, D = feat.shape
    _, N = w.shape

    tb = min(4096, max(8, (B + 7) // 8 * 8))
    b_pad = (B + tb - 1) // tb * tb
    feat_p = jnp.pad(feat, ((0, b_pad - B), (0, 0))) if b_pad != B else feat

    out = pl.pallas_call(
        _fused_kernel,
        out_shape=jax.ShapeDtypeStruct((b_pad, N), jnp.float32),
        grid=(b_pad // tb,),
        in_specs=[
            pl.BlockSpec((tb, D), lambda i: (i, 0)),
            pl.BlockSpec((D, N), lambda i: (0, 0)),
            pl.BlockSpec((1, N), lambda i: (0, 0)),
        ],
        out_specs=pl.BlockSpec((tb, N), lambda i: (i, 0)),
        compiler_params=pltpu.CompilerParams(
            dimension_semantics=("arbitrary",),
        ),
    )(feat_p, w, b.reshape(1, N))

    return out[:B]
